# baseline (speedup 1.0000x reference)
import jax
import jax.numpy as jnp
from jax.experimental import pallas as pl
from jax.experimental.pallas import tpu as pltpu

BM = 256
BN = 2048
PREP_BW = 256


def _prep_kernel(w_ref, m_ref, wm_ref):
    wm_ref[...] = (w_ref[...] * m_ref[...]).astype(jnp.bfloat16)


def _matmul_kernel(x_ref, w_ref, b_ref, o_ref):
    o_ref[...] = (
        jnp.dot(
            x_ref[...].astype(jnp.bfloat16),
            w_ref[...],
            preferred_element_type=jnp.float32,
        )
        + b_ref[...]
    )


def kernel(inputs, w, sparsity_mask, b):
    M, K = inputs.shape
    _, N = w.shape
    wm16 = pl.pallas_call(
        _prep_kernel,
        grid=(K // PREP_BW,),
        in_specs=[
            pl.BlockSpec((PREP_BW, N), lambda i: (i, 0)),
            pl.BlockSpec((PREP_BW, N), lambda i: (i, 0)),
        ],
        out_specs=pl.BlockSpec((PREP_BW, N), lambda i: (i, 0)),
        out_shape=jax.ShapeDtypeStruct((K, N), jnp.bfloat16),
        compiler_params=pltpu.CompilerParams(
            dimension_semantics=("arbitrary",),
        ),
    )(w, sparsity_mask)
    b2d = b.reshape(1, N)
    grid = (N // BN, M // BM)
    out = pl.pallas_call(
        _matmul_kernel,
        grid=grid,
        in_specs=[
            pl.BlockSpec((BM, K), lambda j, i: (i, 0)),
            pl.BlockSpec((K, BN), lambda j, i: (0, j)),
            pl.BlockSpec((1, BN), lambda j, i: (0, j)),
        ],
        out_specs=pl.BlockSpec((BM, BN), lambda j, i: (i, j)),
        out_shape=jax.ShapeDtypeStruct((M, N), jnp.float32),
        compiler_params=pltpu.CompilerParams(
            dimension_semantics=("arbitrary", "arbitrary"),
            vmem_limit_bytes=67000000,
        ),
    )(inputs, wm16, b2d)
    return out


# speedup vs baseline: 1.0406x; 1.0406x over previous
import jax
import jax.numpy as jnp
from jax.experimental import pallas as pl
from jax.experimental.pallas import tpu as pltpu

BM = 512
BN = 2048
PREP_BW = 256


def _prep_kernel(w_ref, m_ref, wm_ref):
    wm_ref[...] = (w_ref[...] * m_ref[...]).astype(jnp.bfloat16)


def _matmul_kernel(x_ref, w_ref, b_ref, o_ref):
    o_ref[...] = (
        jnp.dot(
            x_ref[...].astype(jnp.bfloat16),
            w_ref[...],
            preferred_element_type=jnp.float32,
        )
        + b_ref[...]
    )


def kernel(inputs, w, sparsity_mask, b):
    M, K = inputs.shape
    _, N = w.shape
    wm16 = pl.pallas_call(
        _prep_kernel,
        grid=(K // PREP_BW,),
        in_specs=[
            pl.BlockSpec((PREP_BW, N), lambda i: (i, 0)),
            pl.BlockSpec((PREP_BW, N), lambda i: (i, 0)),
        ],
        out_specs=pl.BlockSpec((PREP_BW, N), lambda i: (i, 0)),
        out_shape=jax.ShapeDtypeStruct((K, N), jnp.bfloat16),
        compiler_params=pltpu.CompilerParams(
            dimension_semantics=("arbitrary",),
        ),
    )(w, sparsity_mask)
    b2d = b.reshape(1, N)
    grid = (N // BN, M // BM)
    out = pl.pallas_call(
        _matmul_kernel,
        grid=grid,
        in_specs=[
            pl.BlockSpec((BM, K), lambda j, i: (i, 0)),
            pl.BlockSpec((K, BN), lambda j, i: (0, j)),
            pl.BlockSpec((1, BN), lambda j, i: (0, j)),
        ],
        out_specs=pl.BlockSpec((BM, BN), lambda j, i: (i, j)),
        out_shape=jax.ShapeDtypeStruct((M, N), jnp.float32),
        compiler_params=pltpu.CompilerParams(
            dimension_semantics=("arbitrary", "arbitrary"),
            vmem_limit_bytes=67000000,
        ),
    )(inputs, wm16, b2d)
    return out
